# baseline (speedup 1.0000x reference)
import functools

import jax
import jax.numpy as jnp
from jax import lax
from jax.experimental import pallas as pl
from jax.experimental.pallas import tpu as pltpu
from jax.experimental.pallas import tpu_sc as plsc

VOCAB = 1000000
EMBED = 64
NCLS = 128
BATCH = 4096
SEQ = 50

NW = 32
N = BATCH * SEQ
ROWS_PER_W = N // NW
CHUNK = 128
NCHUNK = ROWS_PER_W // CHUNK

VB = 2048


def _pt_body(tt_ref, w_ref, b_ref, out_ref):
    tt = tt_ref[...].astype(jnp.bfloat16)
    w = w_ref[...].astype(jnp.bfloat16)
    pt = lax.dot_general(tt, w, (((0,), (1,)), ((), ())),
                         preferred_element_type=jnp.float32)
    out_ref[...] = (pt + b_ref[...]).astype(jnp.bfloat16)


def _pt_matmul(tt, W, b2):
    grid = (VOCAB + VB - 1) // VB
    return pl.pallas_call(
        _pt_body,
        grid=(grid,),
        in_specs=[
            pl.BlockSpec((EMBED, VB), lambda i: (0, i)),
            pl.BlockSpec((NCLS, EMBED), lambda i: (0, 0)),
            pl.BlockSpec((1, NCLS), lambda i: (0, 0)),
        ],
        out_specs=pl.BlockSpec((VB, NCLS), lambda i: (i, 0)),
        out_shape=jax.ShapeDtypeStruct((VOCAB, NCLS), jnp.bfloat16),
    )(tt, W, b2)


@functools.cache
def _build_sc_gather():
    mesh = plsc.VectorSubcoreMesh(core_axis_name="c", subcore_axis_name="s")

    @functools.partial(
        pl.kernel,
        mesh=mesh,
        out_type=jax.ShapeDtypeStruct((N, NCLS), jnp.bfloat16),
        scratch_types=[
            pltpu.VMEM((NCHUNK, CHUNK), jnp.int32),
            pltpu.VMEM((CHUNK, NCLS), jnp.bfloat16),
            pltpu.SemaphoreType.DMA,
        ],
        compiler_params=pltpu.CompilerParams(use_tc_tiling_on_sc=False),
    )
    def _sc_gather(idx_hbm, pt_hbm, out_hbm, idx_v, buf, gsem):
        wid = lax.axis_index("s") * 2 + lax.axis_index("c")
        base = wid * ROWS_PER_W
        pltpu.sync_copy(idx_hbm.at[wid], idx_v)

        def body(j, carry):
            pltpu.async_copy(pt_hbm.at[idx_v.at[j]], buf, gsem).wait()
            pltpu.sync_copy(buf, out_hbm.at[pl.ds(base + j * CHUNK, CHUNK)])
            return carry

        lax.fori_loop(0, NCHUNK, body, 0)

    return _sc_gather


G = 32
R = SEQ * G


def _sm_body(lg_ref, m_ref, mt_ref, out_ref):
    e = jnp.exp(lg_ref[...].astype(jnp.float32))
    s = jnp.dot(m_ref[...], e, preferred_element_type=jnp.float32)
    drep = jnp.dot(mt_ref[...], s, preferred_element_type=jnp.float32)
    o = e / drep
    for g in range(G):
        out_ref[g] = o[g * SEQ:(g + 1) * SEQ, :]


def kernel(text, table, W, b):
    tt = table.T
    b2 = b.reshape(1, NCLS)
    pt = _pt_matmul(tt, W, b2)

    idx = text.reshape(NW, NCHUNK, CHUNK).astype(jnp.int32)
    lg = _build_sc_gather()(idx, pt)

    gid = lax.broadcasted_iota(jnp.int32, (1, R), 1) // SEQ
    m = (gid == lax.broadcasted_iota(jnp.int32, (G, 1), 0)).astype(jnp.float32)
    mt = m.T

    return pl.pallas_call(
        _sm_body,
        grid=(BATCH // G,),
        in_specs=[
            pl.BlockSpec((R, NCLS), lambda i: (i, 0)),
            pl.BlockSpec((G, R), lambda i: (0, 0)),
            pl.BlockSpec((R, G), lambda i: (0, 0)),
        ],
        out_specs=pl.BlockSpec((G, SEQ, NCLS), lambda i: (i, 0, 0)),
        out_shape=jax.ShapeDtypeStruct((BATCH, SEQ, NCLS), jnp.float32),
    )(lg, m, mt)


# speedup vs baseline: 1.9050x; 1.9050x over previous
import functools

import jax
import jax.numpy as jnp
from jax import lax
from jax.experimental import pallas as pl
from jax.experimental.pallas import tpu as pltpu
from jax.experimental.pallas import tpu_sc as plsc

VOCAB = 1000000
EMBED = 64
NCLS = 128
BATCH = 4096
SEQ = 50

NW = 32
N = BATCH * SEQ
ROWS_PER_W = N // NW
CHUNK = 128
NCHUNK = ROWS_PER_W // CHUNK


@functools.cache
def _build_sc_gather():
    mesh = plsc.VectorSubcoreMesh(core_axis_name="c", subcore_axis_name="s")

    @functools.partial(
        pl.kernel,
        mesh=mesh,
        out_type=jax.ShapeDtypeStruct((N, 2 * EMBED), jnp.float32),
        scratch_types=[
            pltpu.VMEM((NCHUNK, CHUNK), jnp.int32),
            pltpu.VMEM((CHUNK, 2 * EMBED), jnp.float32),
            pltpu.SemaphoreType.DMA,
        ],
        compiler_params=pltpu.CompilerParams(use_tc_tiling_on_sc=False),
    )
    def _sc_gather(idx_hbm, table2_hbm, out_hbm, idx_v, buf, gsem):
        wid = lax.axis_index("s") * 2 + lax.axis_index("c")
        base = wid * ROWS_PER_W
        pltpu.sync_copy(idx_hbm.at[wid], idx_v)

        def body(j, carry):
            pltpu.async_copy(table2_hbm.at[idx_v.at[j]], buf, gsem).wait()
            pltpu.sync_copy(buf, out_hbm.at[pl.ds(base + j * CHUNK, CHUNK)])
            return carry

        lax.fori_loop(0, NCHUNK, body, 0)

    return _sc_gather


G = 64
R = SEQ * G


def _tc_body(x_ref, w2_ref, b2_ref, p_ref, m_ref, mt_ref, out_ref):
    x = x_ref[...]
    logits = jnp.dot(x, w2_ref[...], preferred_element_type=jnp.float32)
    e = jnp.exp(logits + b2_ref[...])
    p = p_ref[...].reshape(R, 1)
    e_sel = e[:, :NCLS] + p * (e[:, NCLS:] - e[:, :NCLS])
    s = jnp.dot(m_ref[...], e_sel, preferred_element_type=jnp.float32)
    drep = jnp.dot(mt_ref[...], s, preferred_element_type=jnp.float32)
    o = e_sel / drep
    for g in range(G):
        out_ref[g] = o[g * SEQ:(g + 1) * SEQ, :]


def kernel(text, table, W, b):
    table2 = table.reshape(VOCAB // 2, 2 * EMBED)
    tokens = text.reshape(-1).astype(jnp.int32)
    idx = (tokens >> 1).reshape(NW, NCHUNK, CHUNK)
    parity = (tokens & 1).astype(jnp.float32).reshape(BATCH // G, 1, R)
    embw = _build_sc_gather()(idx, table2)

    wt = W.T
    w2 = jnp.zeros((2 * EMBED, 2 * NCLS), jnp.float32)
    w2 = w2.at[:EMBED, :NCLS].set(wt).at[EMBED:, NCLS:].set(wt)
    b2 = jnp.concatenate([b, b]).reshape(1, 2 * NCLS)
    gid = lax.broadcasted_iota(jnp.int32, (1, R), 1) // SEQ
    m = (gid == lax.broadcasted_iota(jnp.int32, (G, 1), 0)).astype(jnp.float32)
    mt = m.T

    return pl.pallas_call(
        _tc_body,
        grid=(BATCH // G,),
        in_specs=[
            pl.BlockSpec((R, 2 * EMBED), lambda i: (i, 0)),
            pl.BlockSpec((2 * EMBED, 2 * NCLS), lambda i: (0, 0)),
            pl.BlockSpec((1, 2 * NCLS), lambda i: (0, 0)),
            pl.BlockSpec((1, 1, R), lambda i: (i, 0, 0)),
            pl.BlockSpec((G, R), lambda i: (0, 0)),
            pl.BlockSpec((R, G), lambda i: (0, 0)),
        ],
        out_specs=pl.BlockSpec((G, SEQ, NCLS), lambda i: (i, 0, 0)),
        out_shape=jax.ShapeDtypeStruct((BATCH, SEQ, NCLS), jnp.float32),
    )(embw, w2, b2, parity, m, mt)
